# baseline (speedup 1.0000x reference)
import functools

import jax
import jax.numpy as jnp
from jax import lax
from jax.experimental import pallas as pl
from jax.experimental.pallas import tpu as pltpu
from jax.experimental.pallas import tpu_sc as plsc

B = 4096
S = 200
D = 32
C = 10
UA = 104
UBN = 96
NC = 2
NS = 16
NW = NC * NS
CB = B // NW
UB = CB * 2
SP = 208
NBUF = 4
DP = 128

_mesh = plsc.VectorSubcoreMesh(core_axis_name="c", subcore_axis_name="s")


@functools.partial(
    pl.kernel,
    mesh=_mesh,
    compiler_params=pltpu.CompilerParams(use_tc_tiling_on_sc=False),
    out_type=jax.ShapeDtypeStruct((B, D), jnp.float32),
    scratch_types=[
        pltpu.VMEM((CB, SP), jnp.float32),
        pltpu.VMEM((CB, SP), jnp.int32),
        pltpu.VMEM((CB, D), jnp.float32),
    ]
    + [pltpu.VMEM((UA, DP), jnp.float32) for _ in range(NBUF)]
    + [pltpu.SemaphoreType.DMA for _ in range(NBUF)],
)
def _pool(table, xf, out, idx_f, idx_v, pooled_v,
          b0, b1, b2, b3,
          s0, s1, s2, s3):
    bufs = (b0, b1, b2, b3)
    sems = (s0, s1, s2, s3)
    wid = lax.axis_index("s") * NC + lax.axis_index("c")
    bbase = wid * CB

    pltpu.sync_copy(xf.at[pl.ds(bbase, CB)], idx_f.at[:, pl.ds(0, S)])

    def cvt(j, _):
        for k in range(SP // 16):
            idx_v[j, pl.ds(k * 16, 16)] = (
                idx_f[j, pl.ds(k * 16, 16)].astype(jnp.int32))
        return 0

    lax.fori_loop(0, CB, cvt, 0)

    def _unit(u, b):
        j = u // 2
        if b % 2 == 0:
            return idx_v.at[j, pl.ds(0, UA)], bufs[b]
        return idx_v.at[j, pl.ds(UA, UBN)], bufs[b].at[pl.ds(0, UBN)]

    def _start(u, b):
        src, dst = _unit(u, b)
        pltpu.async_copy(table.at[src], dst, sems[b])

    for b in range(NBUF):
        _start(b, b)

    def outer(g, _):
        u0 = NBUF * g
        for jj in range(NBUF // 2):
            jrow = (NBUF // 2) * g + jj
            acc = (jnp.zeros((16,), jnp.float32),) * 4
            for b2 in range(2):
                b = 2 * jj + b2
                u = u0 + b
                src, dst = _unit(u, b)
                pltpu.make_async_copy(table.at[src], dst, sems[b]).wait()
                buf = bufs[b]
                nrows = UA if b % 2 == 0 else UBN

                def red(i, a, buf=buf):
                    a0, a1, a2, a3 = a
                    r = i * 8
                    for q in range(0, 8, 2):
                        a0 = a0 + buf[r + q, pl.ds(0, 16)]
                        a1 = a1 + buf[r + q, pl.ds(16, 16)]
                        a2 = a2 + buf[r + q + 1, pl.ds(0, 16)]
                        a3 = a3 + buf[r + q + 1, pl.ds(16, 16)]
                    return (a0, a1, a2, a3)

                acc = lax.fori_loop(0, nrows // 8, red, acc)

                @pl.when(u + NBUF < UB)
                def _(b=b, u=u):
                    _start(u + NBUF, b)

            pooled_v[jrow, pl.ds(0, 16)] = acc[0] + acc[2]
            pooled_v[jrow, pl.ds(16, 16)] = acc[1] + acc[3]
        return 0

    lax.fori_loop(0, UB // NBUF, outer, 0)
    pltpu.sync_copy(pooled_v, out.at[pl.ds(bbase, CB)])


def _linear_body(p_ref, w_ref, b_ref, o_ref):
    o_ref[...] = (
        jnp.dot(p_ref[...], w_ref[...], preferred_element_type=jnp.float32)
        + b_ref[...]
    )


def _linear(pooled, w, b):
    return pl.pallas_call(
        _linear_body,
        out_shape=jax.ShapeDtypeStruct((B, C), jnp.float32),
        grid=(4,),
        in_specs=[
            pl.BlockSpec((B // 4, D), lambda i: (i, 0)),
            pl.BlockSpec((D, C), lambda i: (0, 0)),
            pl.BlockSpec((1, C), lambda i: (0, 0)),
        ],
        out_specs=pl.BlockSpec((B // 4, C), lambda i: (i, 0)),
    )(pooled, w, b)


def kernel(x, emb_table, fc_w, fc_b):
    tbl_p = jnp.pad(emb_table, ((0, 0), (0, DP - D)))
    pooled = _pool(tbl_p, x.astype(jnp.float32))
    w = fc_w.T.astype(jnp.float32) * jnp.float32(1.0 / S)
    return _linear(pooled, w, fc_b.reshape(1, C).astype(jnp.float32))


# speedup vs baseline: 1.2985x; 1.2985x over previous
import functools

import jax
import jax.numpy as jnp
from jax import lax
from jax.experimental import pallas as pl
from jax.experimental.pallas import tpu as pltpu
from jax.experimental.pallas import tpu_sc as plsc

B = 4096
S = 200
D = 32
C = 10
CP = 16
V = 1000000
DP = 128
UA = 104
UBN = 96
NC = 2
NS = 16
NW = NC * NS
CB = B // NW
UB = CB * 2
SP = 208
NBUF = 4
VBLK = 4096

_mesh = plsc.VectorSubcoreMesh(core_axis_name="c", subcore_axis_name="s")


def _proj_body(t_ref, w_ref, o_ref):
    o_ref[:, 0:CP] = lax.dot_general(
        t_ref[...], w_ref[...], (((0,), (1,)), ((), ())),
        preferred_element_type=jnp.float32)


def _project(tblT, w16):
    return pl.pallas_call(
        _proj_body,
        out_shape=jax.ShapeDtypeStruct((V, DP), jnp.float32),
        grid=(pl.cdiv(V, VBLK),),
        in_specs=[
            pl.BlockSpec((D, VBLK), lambda i: (0, i)),
            pl.BlockSpec((CP, D), lambda i: (0, 0)),
        ],
        out_specs=pl.BlockSpec((VBLK, DP), lambda i: (i, 0)),
    )(tblT, w16)


@functools.partial(
    pl.kernel,
    mesh=_mesh,
    compiler_params=pltpu.CompilerParams(use_tc_tiling_on_sc=False),
    out_type=jax.ShapeDtypeStruct((B, CP), jnp.float32),
    scratch_types=[
        pltpu.VMEM((CB, SP), jnp.float32),
        pltpu.VMEM((CB, SP), jnp.int32),
        pltpu.VMEM((CB, CP), jnp.float32),
        pltpu.VMEM((CP,), jnp.float32),
    ]
    + [pltpu.VMEM((UA, DP), jnp.float32) for _ in range(NBUF)]
    + [pltpu.SemaphoreType.DMA for _ in range(NBUF)],
)
def _pool(proj, xf, b16, out, idx_f, idx_v, pooled_v, bvec,
          b0, b1, b2, b3,
          s0, s1, s2, s3):
    bufs = (b0, b1, b2, b3)
    sems = (s0, s1, s2, s3)
    wid = lax.axis_index("s") * NC + lax.axis_index("c")
    bbase = wid * CB

    pltpu.sync_copy(b16, bvec)
    pltpu.sync_copy(xf.at[pl.ds(bbase, CB)], idx_f.at[:, pl.ds(0, S)])

    def cvt(j, _):
        for k in range(SP // 16):
            idx_v[j, pl.ds(k * 16, 16)] = (
                idx_f[j, pl.ds(k * 16, 16)].astype(jnp.int32))
        return 0

    lax.fori_loop(0, CB, cvt, 0)

    def _unit(u, b):
        j = u // 2
        if b % 2 == 0:
            return idx_v.at[j, pl.ds(0, UA)], bufs[b]
        return idx_v.at[j, pl.ds(UA, UBN)], bufs[b].at[pl.ds(0, UBN)]

    def _start(u, b):
        src, dst = _unit(u, b)
        pltpu.async_copy(proj.at[src], dst, sems[b])

    for b in range(NBUF):
        _start(b, b)

    bias = bvec[...]

    def outer(g, _):
        u0 = NBUF * g
        for jj in range(NBUF // 2):
            jrow = (NBUF // 2) * g + jj
            acc = (jnp.zeros((16,), jnp.float32),) * 4
            for b2 in range(2):
                b = 2 * jj + b2
                u = u0 + b
                src, dst = _unit(u, b)
                pltpu.make_async_copy(proj.at[src], dst, sems[b]).wait()
                buf = bufs[b]
                nrows = UA if b % 2 == 0 else UBN

                def red(i, a, buf=buf):
                    a0, a1, a2, a3 = a
                    r = i * 8
                    a0 = a0 + buf[r, pl.ds(0, 16)]
                    a1 = a1 + buf[r + 1, pl.ds(0, 16)]
                    a2 = a2 + buf[r + 2, pl.ds(0, 16)]
                    a3 = a3 + buf[r + 3, pl.ds(0, 16)]
                    a0 = a0 + buf[r + 4, pl.ds(0, 16)]
                    a1 = a1 + buf[r + 5, pl.ds(0, 16)]
                    a2 = a2 + buf[r + 6, pl.ds(0, 16)]
                    a3 = a3 + buf[r + 7, pl.ds(0, 16)]
                    return (a0, a1, a2, a3)

                acc = lax.fori_loop(0, nrows // 8, red, acc)

                @pl.when(u + NBUF < UB)
                def _(b=b, u=u):
                    _start(u + NBUF, b)

            pooled_v[jrow, pl.ds(0, 16)] = (
                (acc[0] + acc[1]) + (acc[2] + acc[3]) + bias)
        return 0

    lax.fori_loop(0, UB // NBUF, outer, 0)
    pltpu.sync_copy(pooled_v, out.at[pl.ds(bbase, CB)])


def kernel(x, emb_table, fc_w, fc_b):
    w16 = jnp.zeros((CP, D), jnp.float32).at[:C].set(
        fc_w.astype(jnp.float32) * jnp.float32(1.0 / S))
    b16 = jnp.zeros((CP,), jnp.float32).at[:C].set(fc_b.astype(jnp.float32))
    proj = _project(emb_table.T, w16)
    out16 = _pool(proj, x.astype(jnp.float32), b16)
    return out16[:, :C]
